# baseline (speedup 1.0000x reference)
import functools

import jax
import jax.numpy as jnp
from jax import lax
from jax.experimental import pallas as pl
from jax.experimental.pallas import tpu as pltpu
from jax.experimental.pallas import tpu_sc as plsc

NC, NS, L = 2, 16, 16
NW = NC * NS
B = 16384
D = 32
K = 7
NPAIR = K - 1
RPT = B // NW
CHUNK = 128
NCHUNK = RPT // CHUNK
NIDX = K * RPT
PACK = 128 // D


@functools.cache
def _make_sc_dots():
    mesh = plsc.VectorSubcoreMesh(
        core_axis_name="c", subcore_axis_name="s", num_cores=NC, num_subcores=NS
    )
    return pl.kernel(
        _sc_dots_body,
        out_type=jax.ShapeDtypeStruct((NW, NPAIR * RPT), jnp.float32),
        mesh=mesh,
        compiler_params=pltpu.CompilerParams(
            use_tc_tiling_on_sc=False, needs_layout_passes=False
        ),
        scratch_types=[
            pltpu.VMEM((NIDX,), jnp.int32),
            pltpu.VMEM((NIDX,), jnp.int32),
            pltpu.VMEM((CHUNK, 128), jnp.float32),
            pltpu.VMEM((NPAIR, CHUNK, 128), jnp.float32),
            pltpu.VMEM((NPAIR * RPT,), jnp.float32),
            pltpu.SemaphoreType.DMA,
        ],
    )


def _sc_dots_body(batch_hbm, node_hbm, ctx_hbm, out_hbm, idx_v, hi_v, vi_v, ctx_v,
                  dots_v, sem):
    wid = lax.axis_index("s") * NC + lax.axis_index("c")
    pltpu.sync_copy(batch_hbm.at[wid], idx_v)

    def hi_body(j, carry):
        raw = idx_v[pl.ds(j * L, L)]
        hi = lax.shift_left(
            lax.shift_right_logical(raw, SHIFT_T), SHIFT_S
        ) | (raw & (SUB - 1))
        hi_v[pl.ds(j * L, L)] = hi
        return carry

    lax.fori_loop(0, NIDX // L, hi_body, 0)

    iota = lax.iota(jnp.int32, L)

    def chunk_body(i, carry):
        cps = [
            pltpu.async_copy(node_hbm.at[hi_v.at[pl.ds(i * CHUNK, CHUNK)]], vi_v, sem)
        ]
        for c in range(NPAIR):
            off = (c + 1) * RPT + i * CHUNK
            cps.append(
                pltpu.async_copy(
                    ctx_hbm.at[hi_v.at[pl.ds(off, CHUNK)]], ctx_v.at[c], sem
                )
            )
        for cp in cps:
            cp.wait()

        def group_body(g, gcarry):
            rows = g * L + iota
            base = i * CHUNK + g * L
            vi_lo = lax.shift_right_logical(idx_v[pl.ds(base, L)], SHIFT_S - 5) & 96
            acc = [jnp.zeros((L,), jnp.float32) for _ in range(NPAIR)]
            los = [
                lax.shift_right_logical(
                    idx_v[pl.ds((c + 1) * RPT + base, L)], SHIFT_S - 5
                ) & 96
                for c in range(NPAIR)
            ]
            for d in range(D):
                vi_d = plsc.load_gather(vi_v, [rows, vi_lo + d])
                for c in range(NPAIR):
                    ctx_d = plsc.load_gather(
                        ctx_v, [jnp.full((L,), c, jnp.int32), rows, los[c] + d]
                    )
                    acc[c] = acc[c] + vi_d * ctx_d
            dots_v[pl.ds(base, L)] = acc[0]
            for c in range(1, NPAIR):
                dots_v[pl.ds(c * RPT + base, L)] = -acc[c]
            return gcarry

        return lax.fori_loop(0, CHUNK // L, group_body, carry)

    lax.fori_loop(0, NCHUNK, chunk_body, 0)
    pltpu.sync_copy(dots_v, out_hbm.at[wid])


TBLK = 65536
SUB = TBLK // PACK
SHIFT_T = TBLK.bit_length() - 1
SHIFT_S = SUB.bit_length() - 1


def _sel(nn):
    f = lax.broadcasted_iota(jnp.int32, (D, 128), 0)
    k = lax.broadcasted_iota(jnp.int32, (D, 128), 1)
    return jnp.where(k == nn * D + f, 1.0, 0.0).astype(jnp.float32)


def _tc_pack_body(x_ref, o_ref):
    x = x_ref[...]
    acc = jnp.zeros((SUB, 128), jnp.float32)
    for nn in range(PACK):
        acc = acc + lax.dot_general(
            x[:, nn * SUB:(nn + 1) * SUB], _sel(nn), (((0,), (0,)), ((), ())),
            preferred_element_type=jnp.float32,
        )
    o_ref[...] = acc


def _relayout(table_t):
    n = table_t.shape[1]
    grid = (n + TBLK - 1) // TBLK
    return pl.pallas_call(
        _tc_pack_body,
        grid=(grid,),
        in_specs=[pl.BlockSpec((D, TBLK), lambda i: (0, i))],
        out_specs=pl.BlockSpec((SUB, 128), lambda i: (i, 0)),
        out_shape=jax.ShapeDtypeStruct((grid * SUB, 128), jnp.float32),
    )(table_t)


def _tc_loss_body(x_ref, o_ref):
    x = x_ref[...]
    ls = jnp.minimum(x, 0.0) - jnp.log1p(jnp.exp(-jnp.abs(x)))
    o_ref[0, 0] = -jnp.sum(ls)


def kernel(batch, node_embed, context_node_embed):
    idx = batch.astype(jnp.int32)
    idx_t = idx.T.reshape(K, NW, RPT).transpose(1, 0, 2).reshape(NW, NIDX)
    node_r = _relayout(node_embed.T)
    ctx_r = _relayout(context_node_embed.T)
    dots = _make_sc_dots()(idx_t, node_r, ctx_r)
    x = dots.reshape(B * NPAIR // 128, 128)
    loss = pl.pallas_call(
        _tc_loss_body,
        out_shape=jax.ShapeDtypeStruct((1, 1), jnp.float32),
        out_specs=pl.BlockSpec(memory_space=pltpu.SMEM),
    )(x)
    return loss[0, 0]


# speedup vs baseline: 1.0583x; 1.0583x over previous
import functools

import jax
import jax.numpy as jnp
from jax import lax
from jax.experimental import pallas as pl
from jax.experimental.pallas import tpu as pltpu
from jax.experimental.pallas import tpu_sc as plsc

NC, NS, L = 2, 16, 16
NW = NC * NS
B = 16384
D = 32
K = 7
NPAIR = K - 1
RPT = B // NW
CHUNK = 64
NCHUNK = RPT // CHUNK
NIDX = K * RPT
GPC = CHUNK // L
PACK = 128 // D


@functools.cache
def _make_sc_dots():
    mesh = plsc.VectorSubcoreMesh(
        core_axis_name="c", subcore_axis_name="s", num_cores=NC, num_subcores=NS
    )
    return pl.kernel(
        _sc_dots_body,
        out_type=jax.ShapeDtypeStruct((NW, NPAIR * RPT), jnp.float32),
        mesh=mesh,
        compiler_params=pltpu.CompilerParams(
            use_tc_tiling_on_sc=False, needs_layout_passes=False
        ),
        scratch_types=[
            pltpu.VMEM((NIDX,), jnp.int32),
            pltpu.VMEM((NIDX,), jnp.int32),
            pltpu.VMEM((2, CHUNK, 128), jnp.float32),
            pltpu.VMEM((2, NPAIR * CHUNK, 128), jnp.float32),
            pltpu.VMEM((NPAIR * RPT,), jnp.float32),
            pltpu.SemaphoreType.DMA,
            pltpu.SemaphoreType.DMA,
        ],
    )


def _sc_dots_body(batch_hbm, node_hbm, ctx_hbm, out_hbm, idx_v, hi_v, vi_v, cx_v,
                  dots_v, sem0, sem1):
    wid = lax.axis_index("s") * NC + lax.axis_index("c")
    pltpu.sync_copy(batch_hbm.at[wid], idx_v)

    def hi_body(j, carry):
        raw = idx_v[pl.ds(j * L, L)]
        hi = lax.shift_left(
            lax.shift_right_logical(raw, SHIFT_T), SHIFT_S
        ) | (raw & (SUB - 1))
        hi_v[pl.ds(j * L, L)] = hi
        return carry

    lax.fori_loop(0, NIDX // L, hi_body, 0)

    iota = lax.iota(jnp.int32, L)
    sems = (sem0, sem1)

    def issue(i):
        base = i * K * CHUNK
        slot = i & 1
        return (
            pltpu.async_copy(
                node_hbm.at[hi_v.at[pl.ds(base, CHUNK)]], vi_v.at[slot], sems[slot]
            ),
            pltpu.async_copy(
                ctx_hbm.at[hi_v.at[pl.ds(base + CHUNK, NPAIR * CHUNK)]],
                cx_v.at[slot],
                sems[slot],
            ),
        )

    def compute(i):
        slot = jnp.int32(i & 1)

        def group_body(g, gcarry):
            rows = g * L + iota
            ibase = i * K * CHUNK + g * L
            obase = i * CHUNK + g * L
            slot_v = jnp.full((L,), i & 1, jnp.int32)
            vi_lo = lax.shift_right_logical(idx_v[pl.ds(ibase, L)], SHIFT_S - 5) & 96
            acc = [jnp.zeros((L,), jnp.float32) for _ in range(NPAIR)]
            los = [
                lax.shift_right_logical(
                    idx_v[pl.ds(ibase + (c + 1) * CHUNK, L)], SHIFT_S - 5
                ) & 96
                for c in range(NPAIR)
            ]
            for d in range(D):
                vi_d = plsc.load_gather(vi_v, [slot_v, rows, vi_lo + d])
                for c in range(NPAIR):
                    ctx_d = plsc.load_gather(
                        cx_v, [slot_v, c * CHUNK + rows, los[c] + d]
                    )
                    acc[c] = acc[c] + vi_d * ctx_d
            dots_v[pl.ds(obase, L)] = acc[0]
            for c in range(1, NPAIR):
                dots_v[pl.ds(c * RPT + obase, L)] = -acc[c]
            return gcarry

        lax.fori_loop(0, GPC, group_body, 0)
        del slot

    pending = issue(0)
    for i in range(NCHUNK):
        nxt = issue(i + 1) if i + 1 < NCHUNK else None
        for cp in pending:
            cp.wait()
        compute(i)
        pending = nxt

    pltpu.sync_copy(dots_v, out_hbm.at[wid])


TBLK = 32768
SUB = TBLK // PACK
SHIFT_T = TBLK.bit_length() - 1
SHIFT_S = SUB.bit_length() - 1


def _sel(nn):
    f = lax.broadcasted_iota(jnp.int32, (D, 128), 0)
    k = lax.broadcasted_iota(jnp.int32, (D, 128), 1)
    return jnp.where(k == nn * D + f, 1.0, 0.0).astype(jnp.float32)


def _tc_pack_body(x_ref, o_ref):
    x = x_ref[...]
    acc = jnp.zeros((SUB, 128), jnp.float32)
    for nn in range(PACK):
        acc = acc + lax.dot_general(
            x[:, nn * SUB:(nn + 1) * SUB], _sel(nn), (((0,), (0,)), ((), ())),
            preferred_element_type=jnp.float32,
        )
    o_ref[...] = acc


def _relayout(table_t):
    n = table_t.shape[1]
    grid = (n + TBLK - 1) // TBLK
    return pl.pallas_call(
        _tc_pack_body,
        grid=(grid,),
        in_specs=[pl.BlockSpec((D, TBLK), lambda i: (0, i))],
        out_specs=pl.BlockSpec((SUB, 128), lambda i: (i, 0)),
        out_shape=jax.ShapeDtypeStruct((grid * SUB, 128), jnp.float32),
    )(table_t)


def _tc_loss_body(x_ref, o_ref):
    x = x_ref[...]
    ls = jnp.minimum(x, 0.0) - jnp.log1p(jnp.exp(-jnp.abs(x)))
    o_ref[0, 0] = -jnp.sum(ls)


def kernel(batch, node_embed, context_node_embed):
    idx = batch.astype(jnp.int32)
    idx_t = (
        idx.T.reshape(K, NW, NCHUNK, CHUNK)
        .transpose(1, 2, 0, 3)
        .reshape(NW, NIDX)
    )
    node_r = _relayout(node_embed.T)
    ctx_r = _relayout(context_node_embed.T)
    dots = _make_sc_dots()(idx_t, node_r, ctx_r)
    x = dots.reshape(B * NPAIR // 128, 128)
    loss = pl.pallas_call(
        _tc_loss_body,
        out_shape=jax.ShapeDtypeStruct((1, 1), jnp.float32),
        out_specs=pl.BlockSpec(memory_space=pltpu.SMEM),
    )(x)
    return loss[0, 0]


# speedup vs baseline: 1.5805x; 1.4935x over previous
import functools

import jax
import jax.numpy as jnp
from jax import lax
from jax.experimental import pallas as pl
from jax.experimental.pallas import tpu as pltpu
from jax.experimental.pallas import tpu_sc as plsc

NC, NS, L = 2, 16, 16
NW = NC * NS
B = 16384
D = 32
K = 7
NPAIR = K - 1
RPT = B // NW
CHUNK = 64
NCHUNK = RPT // CHUNK
NIDX = K * RPT
GPC = CHUNK // L
PACK = 128 // D


@functools.cache
def _make_sc_dots():
    mesh = plsc.VectorSubcoreMesh(
        core_axis_name="c", subcore_axis_name="s", num_cores=NC, num_subcores=NS
    )
    return pl.kernel(
        _sc_dots_body,
        out_type=jax.ShapeDtypeStruct((NW, NPAIR * RPT), jnp.float32),
        mesh=mesh,
        compiler_params=pltpu.CompilerParams(
            use_tc_tiling_on_sc=False, needs_layout_passes=False
        ),
        scratch_types=[
            pltpu.VMEM((NIDX,), jnp.int32),
            pltpu.VMEM((NIDX,), jnp.int32),
            pltpu.VMEM((2, CHUNK, 128), jnp.float32),
            pltpu.VMEM((2, NPAIR * CHUNK, 128), jnp.float32),
            pltpu.VMEM((NPAIR * RPT,), jnp.float32),
            pltpu.SemaphoreType.DMA,
            pltpu.SemaphoreType.DMA,
        ],
    )


def _sc_dots_body(batch_hbm, node_hbm, ctx_hbm, out_hbm, idx_v, hi_v, vi_v, cx_v,
                  dots_v, sem0, sem1):
    wid = lax.axis_index("s") * NC + lax.axis_index("c")
    pltpu.sync_copy(batch_hbm.at[wid], idx_v)

    def hi_body(j, carry):
        raw = idx_v[pl.ds(j * L, L)]
        hi = lax.shift_left(
            lax.shift_right_logical(raw, SHIFT_T), SHIFT_S
        ) | (raw & (SUB - 1))
        hi_v[pl.ds(j * L, L)] = hi
        return carry

    lax.fori_loop(0, NIDX // L, hi_body, 0)

    iota = lax.iota(jnp.int32, L)
    sems = (sem0, sem1)

    def issue(i):
        base = i * K * CHUNK
        slot = i & 1
        return (
            pltpu.async_copy(
                node_hbm.at[hi_v.at[pl.ds(base, CHUNK)]], vi_v.at[slot], sems[slot]
            ),
            pltpu.async_copy(
                ctx_hbm.at[hi_v.at[pl.ds(base + CHUNK, NPAIR * CHUNK)]],
                cx_v.at[slot],
                sems[slot],
            ),
        )

    def compute(i):
        slot = jnp.int32(i & 1)

        def group_body(g, gcarry):
            rows = g * L + iota
            ibase = i * K * CHUNK + g * L
            obase = i * CHUNK + g * L
            slot_v = jnp.full((L,), i & 1, jnp.int32)
            vi_lo = lax.shift_right_logical(idx_v[pl.ds(ibase, L)], SHIFT_S - 5) & 96
            acc = [jnp.zeros((L,), jnp.float32) for _ in range(NPAIR)]
            los = [
                lax.shift_right_logical(
                    idx_v[pl.ds(ibase + (c + 1) * CHUNK, L)], SHIFT_S - 5
                ) & 96
                for c in range(NPAIR)
            ]
            for d in range(D):
                vi_d = plsc.load_gather(vi_v, [slot_v, rows, vi_lo + d])
                for c in range(NPAIR):
                    ctx_d = plsc.load_gather(
                        cx_v, [slot_v, c * CHUNK + rows, los[c] + d]
                    )
                    acc[c] = acc[c] + vi_d * ctx_d
            dots_v[pl.ds(obase, L)] = acc[0]
            for c in range(1, NPAIR):
                dots_v[pl.ds(c * RPT + obase, L)] = -acc[c]
            return gcarry

        lax.fori_loop(0, GPC, group_body, 0)
        del slot

    pending = issue(0)
    for i in range(NCHUNK):
        nxt = issue(i + 1) if i + 1 < NCHUNK else None
        for cp in pending:
            cp.wait()
        compute(i)
        pending = nxt

    pltpu.sync_copy(dots_v, out_hbm.at[wid])


TBLK = 32768
SUB = TBLK // PACK
SHIFT_T = TBLK.bit_length() - 1
SHIFT_S = SUB.bit_length() - 1


def _tc_pack_body(x_ref, o_ref):
    x = x_ref[...]
    xx = jnp.concatenate(
        [x[:, nn * SUB:(nn + 1) * SUB] for nn in range(PACK)], axis=0
    )
    r = lax.broadcasted_iota(jnp.int32, (128, 128), 0)
    c = lax.broadcasted_iota(jnp.int32, (128, 128), 1)
    eye = jnp.where(r == c, 1.0, 0.0).astype(jnp.float32)
    o_ref[...] = lax.dot_general(
        xx, eye, (((0,), (0,)), ((), ())), preferred_element_type=jnp.float32
    )


def _relayout(table_t):
    n = table_t.shape[1]
    grid = (n + TBLK - 1) // TBLK
    return pl.pallas_call(
        _tc_pack_body,
        grid=(grid,),
        in_specs=[pl.BlockSpec((D, TBLK), lambda i: (0, i))],
        out_specs=pl.BlockSpec((SUB, 128), lambda i: (i, 0)),
        out_shape=jax.ShapeDtypeStruct((grid * SUB, 128), jnp.float32),
    )(table_t)


def _tc_loss_body(x_ref, o_ref):
    x = x_ref[...]
    ls = jnp.minimum(x, 0.0) - jnp.log1p(jnp.exp(-jnp.abs(x)))
    o_ref[0, 0] = -jnp.sum(ls)


def kernel(batch, node_embed, context_node_embed):
    idx = batch.astype(jnp.int32)
    idx_t = (
        idx.T.reshape(K, NW, NCHUNK, CHUNK)
        .transpose(1, 2, 0, 3)
        .reshape(NW, NIDX)
    )
    node_r = _relayout(node_embed.T)
    ctx_r = _relayout(context_node_embed.T)
    dots = _make_sc_dots()(idx_t, node_r, ctx_r)
    x = dots.reshape(B * NPAIR // 128, 128)
    loss = pl.pallas_call(
        _tc_loss_body,
        out_shape=jax.ShapeDtypeStruct((1, 1), jnp.float32),
        out_specs=pl.BlockSpec(memory_space=pltpu.SMEM),
    )(x)
    return loss[0, 0]


# speedup vs baseline: 1.5865x; 1.0038x over previous
import functools

import jax
import jax.numpy as jnp
from jax import lax
from jax.experimental import pallas as pl
from jax.experimental.pallas import tpu as pltpu
from jax.experimental.pallas import tpu_sc as plsc

NC, NS, L = 2, 16, 16
NW = NC * NS
B = 16384
D = 32
K = 7
NPAIR = K - 1
RPT = B // NW
CHUNK = 64
NCHUNK = RPT // CHUNK
NIDX = K * RPT
GPC = CHUNK // L
PACK = 128 // D


@functools.cache
def _make_sc_dots():
    mesh = plsc.VectorSubcoreMesh(
        core_axis_name="c", subcore_axis_name="s", num_cores=NC, num_subcores=NS
    )
    return pl.kernel(
        _sc_dots_body,
        out_type=jax.ShapeDtypeStruct((NW, NPAIR * RPT), jnp.float32),
        mesh=mesh,
        compiler_params=pltpu.CompilerParams(
            use_tc_tiling_on_sc=False, needs_layout_passes=False
        ),
        scratch_types=[
            pltpu.VMEM((NIDX,), jnp.int32),
            pltpu.VMEM((NIDX,), jnp.int32),
            pltpu.VMEM((2, CHUNK, 128), jnp.float32),
            pltpu.VMEM((2, NPAIR * CHUNK, 128), jnp.float32),
            pltpu.VMEM((NPAIR * RPT,), jnp.float32),
            pltpu.SemaphoreType.DMA,
            pltpu.SemaphoreType.DMA,
        ],
    )


def _sc_dots_body(batch_hbm, node_hbm, ctx_hbm, out_hbm, idx_v, hi_v, vi_v, cx_v,
                  dots_v, sem0, sem1):
    wid = lax.axis_index("s") * NC + lax.axis_index("c")
    pltpu.sync_copy(batch_hbm.at[wid], idx_v)

    def hi_body(j, carry):
        raw = idx_v[pl.ds(j * L, L)]
        hi = lax.shift_left(
            lax.shift_right_logical(raw, SHIFT_T), SHIFT_S
        ) | (raw & (SUB - 1))
        hi_v[pl.ds(j * L, L)] = hi
        return carry

    lax.fori_loop(0, NIDX // L, hi_body, 0)

    iota = lax.iota(jnp.int32, L)
    sems = (sem0, sem1)

    def issue(i):
        base = i * K * CHUNK
        slot = i & 1
        return (
            pltpu.async_copy(
                node_hbm.at[hi_v.at[pl.ds(base, CHUNK)]], vi_v.at[slot], sems[slot]
            ),
            pltpu.async_copy(
                ctx_hbm.at[hi_v.at[pl.ds(base + CHUNK, NPAIR * CHUNK)]],
                cx_v.at[slot],
                sems[slot],
            ),
        )

    def compute(i):
        slot = jnp.int32(i & 1)

        def group_body(g, gcarry):
            rows = g * L + iota
            ibase = i * K * CHUNK + g * L
            obase = i * CHUNK + g * L
            slot_v = jnp.full((L,), i & 1, jnp.int32)
            vi_lo = lax.shift_right_logical(idx_v[pl.ds(ibase, L)], SHIFT_S - 5) & 96
            acc = [jnp.zeros((L,), jnp.float32) for _ in range(NPAIR)]
            los = [
                lax.shift_right_logical(
                    idx_v[pl.ds(ibase + (c + 1) * CHUNK, L)], SHIFT_S - 5
                ) & 96
                for c in range(NPAIR)
            ]
            for d in range(D):
                vi_d = plsc.load_gather(vi_v, [slot_v, rows, vi_lo + d])
                for c in range(NPAIR):
                    ctx_d = plsc.load_gather(
                        cx_v, [slot_v, c * CHUNK + rows, los[c] + d]
                    )
                    acc[c] = acc[c] + vi_d * ctx_d
            dots_v[pl.ds(obase, L)] = acc[0]
            for c in range(1, NPAIR):
                dots_v[pl.ds(c * RPT + obase, L)] = -acc[c]
            return gcarry

        lax.fori_loop(0, GPC, group_body, 0)
        del slot

    pending = issue(0)
    for i in range(NCHUNK):
        nxt = issue(i + 1) if i + 1 < NCHUNK else None
        for cp in pending:
            cp.wait()
        compute(i)
        pending = nxt

    pltpu.sync_copy(dots_v, out_hbm.at[wid])


TBLK = 65536
SUB = TBLK // PACK
SHIFT_T = TBLK.bit_length() - 1
SHIFT_S = SUB.bit_length() - 1


def _tc_pack_body(x_ref, o_ref):
    x = x_ref[...]
    xx = jnp.concatenate(
        [x[:, nn * SUB:(nn + 1) * SUB] for nn in range(PACK)], axis=0
    )
    r = lax.broadcasted_iota(jnp.int32, (128, 128), 0)
    c = lax.broadcasted_iota(jnp.int32, (128, 128), 1)
    eye = jnp.where(r == c, 1.0, 0.0).astype(jnp.float32)
    o_ref[...] = lax.dot_general(
        xx, eye, (((0,), (0,)), ((), ())), preferred_element_type=jnp.float32
    )


def _relayout(table_t):
    n = table_t.shape[1]
    grid = (n + TBLK - 1) // TBLK
    return pl.pallas_call(
        _tc_pack_body,
        grid=(grid,),
        in_specs=[pl.BlockSpec((D, TBLK), lambda i: (0, i))],
        out_specs=pl.BlockSpec((SUB, 128), lambda i: (i, 0)),
        out_shape=jax.ShapeDtypeStruct((grid * SUB, 128), jnp.float32),
    )(table_t)


def _tc_loss_body(x_ref, o_ref):
    x = x_ref[...]
    ls = jnp.minimum(x, 0.0) - jnp.log1p(jnp.exp(-jnp.abs(x)))
    o_ref[0, 0] = -jnp.sum(ls)


def kernel(batch, node_embed, context_node_embed):
    idx = batch.astype(jnp.int32)
    idx_t = (
        idx.T.reshape(K, NW, NCHUNK, CHUNK)
        .transpose(1, 2, 0, 3)
        .reshape(NW, NIDX)
    )
    node_r = _relayout(node_embed.T)
    ctx_r = _relayout(context_node_embed.T)
    dots = _make_sc_dots()(idx_t, node_r, ctx_r)
    x = dots.reshape(B * NPAIR // 128, 128)
    loss = pl.pallas_call(
        _tc_loss_body,
        out_shape=jax.ShapeDtypeStruct((1, 1), jnp.float32),
        out_specs=pl.BlockSpec(memory_space=pltpu.SMEM),
    )(x)
    return loss[0, 0]


# speedup vs baseline: 1.6219x; 1.0223x over previous
import functools

import jax
import jax.numpy as jnp
from jax import lax
from jax.experimental import pallas as pl
from jax.experimental.pallas import tpu as pltpu
from jax.experimental.pallas import tpu_sc as plsc

NC, NS, L = 2, 16, 16
NW = NC * NS
B = 16384
D = 32
K = 7
NPAIR = K - 1
RPT = B // NW
CHUNK = 128
NCHUNK = RPT // CHUNK
NIDX = K * RPT
GPC = CHUNK // L
PACK = 128 // D


@functools.cache
def _make_sc_dots():
    mesh = plsc.VectorSubcoreMesh(
        core_axis_name="c", subcore_axis_name="s", num_cores=NC, num_subcores=NS
    )
    return pl.kernel(
        _sc_dots_body,
        out_type=jax.ShapeDtypeStruct((NW, NPAIR * RPT), jnp.float32),
        mesh=mesh,
        compiler_params=pltpu.CompilerParams(
            use_tc_tiling_on_sc=False, needs_layout_passes=False
        ),
        scratch_types=[
            pltpu.VMEM((NIDX,), jnp.int32),
            pltpu.VMEM((NIDX,), jnp.int32),
            pltpu.VMEM((2, CHUNK, D), jnp.float32),
            pltpu.VMEM((2, NPAIR * CHUNK, D), jnp.float32),
            pltpu.VMEM((NPAIR * RPT,), jnp.float32),
            pltpu.SemaphoreType.DMA,
            pltpu.SemaphoreType.DMA,
        ],
    )


def _sc_dots_body(batch_hbm, node_hbm, ctx_hbm, out_hbm, idx_v, hi_v, vi_v, cx_v,
                  dots_v, sem0, sem1):
    wid = lax.axis_index("s") * NC + lax.axis_index("c")
    pltpu.sync_copy(batch_hbm.at[wid], idx_v)

    def hi_body(j, carry):
        raw = idx_v[pl.ds(j * L, L)]
        hi = lax.shift_left(
            lax.shift_right_logical(raw, SHIFT_T), SHIFT_S
        ) | (raw & (SUB - 1))
        hi_v[pl.ds(j * L, L)] = lax.shift_left(hi, 2) | (
            lax.shift_right_logical(raw, SHIFT_S) & 3
        )
        return carry

    lax.fori_loop(0, NIDX // L, hi_body, 0)

    iota = lax.iota(jnp.int32, L)
    sems = (sem0, sem1)

    def issue(i):
        base = i * K * CHUNK
        slot = i & 1
        return (
            pltpu.async_copy(
                node_hbm.at[hi_v.at[pl.ds(base, CHUNK)]], vi_v.at[slot], sems[slot]
            ),
            pltpu.async_copy(
                ctx_hbm.at[hi_v.at[pl.ds(base + CHUNK, NPAIR * CHUNK)]],
                cx_v.at[slot],
                sems[slot],
            ),
        )

    def compute(i):
        slot = jnp.int32(i & 1)

        def group_body(g, gcarry):
            rows = g * L + iota
            ibase = i * K * CHUNK + g * L
            obase = i * CHUNK + g * L
            slot_v = jnp.full((L,), i & 1, jnp.int32)
            acc = [jnp.zeros((L,), jnp.float32) for _ in range(NPAIR)]
            for d in range(D):
                cold = jnp.full((L,), d, jnp.int32)
                vi_d = plsc.load_gather(vi_v, [slot_v, rows, cold])
                for c in range(NPAIR):
                    ctx_d = plsc.load_gather(
                        cx_v, [slot_v, c * CHUNK + rows, cold]
                    )
                    acc[c] = acc[c] + vi_d * ctx_d
            dots_v[pl.ds(obase, L)] = acc[0]
            for c in range(1, NPAIR):
                dots_v[pl.ds(c * RPT + obase, L)] = -acc[c]
            return gcarry

        lax.fori_loop(0, GPC, group_body, 0)
        del slot

    pending = issue(0)
    for i in range(NCHUNK):
        nxt = issue(i + 1) if i + 1 < NCHUNK else None
        for cp in pending:
            cp.wait()
        compute(i)
        pending = nxt

    pltpu.sync_copy(dots_v, out_hbm.at[wid])


TBLK = 65536
SUB = TBLK // PACK
SHIFT_T = TBLK.bit_length() - 1
SHIFT_S = SUB.bit_length() - 1


def _tc_pack_body(x_ref, o_ref):
    x = x_ref[...]
    xx = jnp.concatenate(
        [x[:, nn * SUB:(nn + 1) * SUB] for nn in range(PACK)], axis=0
    )
    r = lax.broadcasted_iota(jnp.int32, (128, 128), 0)
    c = lax.broadcasted_iota(jnp.int32, (128, 128), 1)
    eye = jnp.where(r == c, 1.0, 0.0).astype(jnp.float32)
    o_ref[...] = lax.dot_general(
        xx, eye, (((0,), (0,)), ((), ())), preferred_element_type=jnp.float32
    )


def _relayout(table_t):
    n = table_t.shape[1]
    grid = (n + TBLK - 1) // TBLK
    return pl.pallas_call(
        _tc_pack_body,
        grid=(grid,),
        in_specs=[pl.BlockSpec((D, TBLK), lambda i: (0, i))],
        out_specs=pl.BlockSpec((SUB, 128), lambda i: (i, 0)),
        out_shape=jax.ShapeDtypeStruct((grid * SUB, 128), jnp.float32),
    )(table_t)


def _tc_loss_body(x_ref, o_ref):
    x = x_ref[...]
    ls = jnp.minimum(x, 0.0) - jnp.log1p(jnp.exp(-jnp.abs(x)))
    o_ref[0, 0] = -jnp.sum(ls)


def kernel(batch, node_embed, context_node_embed):
    idx = batch.astype(jnp.int32)
    idx_t = (
        idx.T.reshape(K, NW, NCHUNK, CHUNK)
        .transpose(1, 2, 0, 3)
        .reshape(NW, NIDX)
    )
    node_r = _relayout(node_embed.T).reshape(-1, D)
    ctx_r = _relayout(context_node_embed.T).reshape(-1, D)
    dots = _make_sc_dots()(idx_t, node_r, ctx_r)
    x = dots.reshape(B * NPAIR // 128, 128)
    loss = pl.pallas_call(
        _tc_loss_body,
        out_shape=jax.ShapeDtypeStruct((1, 1), jnp.float32),
        out_specs=pl.BlockSpec(memory_space=pltpu.SMEM),
    )(x)
    return loss[0, 0]
